# baseline (speedup 1.0000x reference)
import functools
import math

import jax
import jax.numpy as jnp
from jax.experimental import pallas as pl
from jax.experimental.pallas import tpu as pltpu


def _lstm3_kernel(x_ref,
                  wih0_ref, b0_ref, whhf0_ref, whhb0_ref,
                  wih1_ref, b1_ref, whhf1_ref, whhb1_ref,
                  wih2_ref, b2_ref, whhf2_ref, whhb2_ref,
                  o_ref, scr_ref, *, T, Bh, G):
    def cell(gates, c):
        i = jax.nn.sigmoid(gates[:, 0 * G:1 * G])
        f = jax.nn.sigmoid(gates[:, 1 * G:2 * G])
        g = jnp.tanh(gates[:, 2 * G:3 * G])
        o = jax.nn.sigmoid(gates[:, 3 * G:4 * G])
        c_new = f * c + i * g
        return o * jnp.tanh(c_new), c_new

    def layer(x2d, wih_ref, b_ref, whhf_ref, whhb_ref, out_ref, out_3d):
        xp = jnp.dot(x2d, wih_ref[...],
                     preferred_element_type=jnp.float32) + b_ref[...]
        whhf = whhf_ref[...]
        whhb = whhb_ref[...]
        h_f = jnp.zeros((Bh, G), jnp.float32)
        c_f = jnp.zeros((Bh, G), jnp.float32)
        h_b = jnp.zeros((Bh, G), jnp.float32)
        c_b = jnp.zeros((Bh, G), jnp.float32)
        for s in range(T):
            tf = s
            tb = T - 1 - s
            rec_f = jnp.dot(h_f, whhf, preferred_element_type=jnp.float32)
            rec_b = jnp.dot(h_b, whhb, preferred_element_type=jnp.float32)
            h_f, c_f = cell(xp[tf * Bh:(tf + 1) * Bh, 0:4 * G] + rec_f, c_f)
            h_b, c_b = cell(xp[tb * Bh:(tb + 1) * Bh, 4 * G:8 * G] + rec_b, c_b)
            if out_3d:
                out_ref[tf, :, 0:G] = h_f
                out_ref[tb, :, G:2 * G] = h_b
            else:
                out_ref[tf * Bh:(tf + 1) * Bh, 0:G] = h_f
                out_ref[tb * Bh:(tb + 1) * Bh, G:2 * G] = h_b

    In0 = x_ref.shape[-1]
    x2d = x_ref[...].reshape(T * Bh, In0)
    layer(x2d, wih0_ref, b0_ref, whhf0_ref, whhb0_ref, scr_ref, False)
    x2d = scr_ref[...]
    layer(x2d, wih1_ref, b1_ref, whhf1_ref, whhb1_ref, scr_ref, False)
    x2d = scr_ref[...]
    layer(x2d, wih2_ref, b2_ref, whhf2_ref, whhb2_ref, o_ref, True)


def _lstm3(xT, lparams, T, B, G):
    In0 = xT.shape[-1]
    flat = []
    wspecs = []
    for (wih, bias, whhf, whhb) in lparams:
        flat += [wih, bias, whhf, whhb]
        wspecs += [
            pl.BlockSpec(wih.shape, lambda i: (0, 0)),
            pl.BlockSpec(bias.shape, lambda i: (0, 0)),
            pl.BlockSpec(whhf.shape, lambda i: (0, 0)),
            pl.BlockSpec(whhb.shape, lambda i: (0, 0)),
        ]

    kern = functools.partial(_lstm3_kernel, T=T, Bh=B, G=G)
    return pl.pallas_call(
        kern,
        out_shape=jax.ShapeDtypeStruct((T, B, 2 * G), jnp.float32),
        grid=(1,),
        in_specs=[pl.BlockSpec((T, B, In0), lambda i: (0, 0, 0))] + wspecs,
        out_specs=pl.BlockSpec((T, B, 2 * G), lambda i: (0, 0, 0)),
        scratch_shapes=[pltpu.VMEM((T * B, 2 * G), jnp.float32)],
        compiler_params=pltpu.CompilerParams(
            dimension_semantics=("arbitrary",)),
    )(xT, *flat)


def _enc2_kernel(x_ref,
                 wqkv0_ref, bqkv0_ref, wo0_ref, bo0_ref,
                 w10_ref, b10_ref, w20_ref, b20_ref, ln0_ref,
                 wqkv1_ref, bqkv1_ref, wo1_ref, bo1_ref,
                 w11_ref, b11_ref, w21_ref, b21_ref, ln1_ref,
                 o_ref, *, num_heads, L):
    E = x_ref.shape[-1]
    d = E // num_heads
    scale = 1.0 / math.sqrt(d)

    def ln(z, g, b):
        mu = jnp.mean(z, axis=-1, keepdims=True)
        var = jnp.mean(jnp.square(z - mu), axis=-1, keepdims=True)
        return (z - mu) * jax.lax.rsqrt(var + 1e-5) * g + b

    def enc_layer(x, wqkv_ref, bqkv_ref, wo_ref, bo_ref,
                  w1_ref, b1_ref, w2_ref, b2_ref, ln_ref, final_norm):
        qkv = jnp.dot(x, wqkv_ref[...],
                      preferred_element_type=jnp.float32) + bqkv_ref[...]
        ctxs = []
        for hh in range(num_heads):
            qh = qkv[:, hh * d:(hh + 1) * d]
            kh = qkv[:, E + hh * d:E + (hh + 1) * d]
            vh = qkv[:, 2 * E + hh * d:2 * E + (hh + 1) * d]
            s = jnp.dot(qh, kh.T, preferred_element_type=jnp.float32) * scale
            s = s - jnp.max(s, axis=-1, keepdims=True)
            p = jnp.exp(s)
            p = p * pl.reciprocal(jnp.sum(p, axis=-1, keepdims=True), approx=True)
            ctxs.append(jnp.dot(p, vh, preferred_element_type=jnp.float32))
        ctx = jnp.concatenate(ctxs, axis=1)
        attn = jnp.dot(ctx, wo_ref[...],
                       preferred_element_type=jnp.float32) + bo_ref[...]
        lnp = ln_ref[...]
        y = ln(x + attn, lnp[0:1, :], lnp[1:2, :])
        ff = jnp.maximum(
            jnp.dot(y, w1_ref[...],
                    preferred_element_type=jnp.float32) + b1_ref[...], 0.0)
        ff = jnp.dot(ff, w2_ref[...],
                     preferred_element_type=jnp.float32) + b2_ref[...]
        y = ln(y + ff, lnp[2:3, :], lnp[3:4, :])
        if final_norm:
            y = ln(y, lnp[4:5, :], lnp[5:6, :])
        return y

    x = x_ref[...]
    x = enc_layer(x, wqkv0_ref, bqkv0_ref, wo0_ref, bo0_ref,
                  w10_ref, b10_ref, w20_ref, b20_ref, ln0_ref, False)
    x = enc_layer(x, wqkv1_ref, bqkv1_ref, wo1_ref, bo1_ref,
                  w11_ref, b11_ref, w21_ref, b21_ref, ln1_ref, True)
    o_ref[...] = x


def _enc2(h2d, eparams, num_heads, L):
    M, E = h2d.shape
    n_groups = M // L

    flat = []
    wspecs = []
    for p in eparams:
        for a in p:
            flat.append(a)
            wspecs.append(pl.BlockSpec(a.shape, lambda i: (0, 0)))

    kern = functools.partial(_enc2_kernel, num_heads=num_heads, L=L)
    return pl.pallas_call(
        kern,
        out_shape=jax.ShapeDtypeStruct((M, E), jnp.float32),
        grid=(n_groups,),
        in_specs=[pl.BlockSpec((L, E), lambda i: (i, 0))] + wspecs,
        out_specs=pl.BlockSpec((L, E), lambda i: (i, 0)),
        compiler_params=pltpu.CompilerParams(
            dimension_semantics=("arbitrary",)),
    )(h2d, *flat)


def kernel(lstm0_fwd_w_ih, lstm0_fwd_w_hh, lstm0_fwd_b_ih, lstm0_fwd_b_hh,
           lstm0_bwd_w_ih, lstm0_bwd_w_hh, lstm0_bwd_b_ih, lstm0_bwd_b_hh,
           lstm1_fwd_w_ih, lstm1_fwd_w_hh, lstm1_fwd_b_ih, lstm1_fwd_b_hh,
           lstm1_bwd_w_ih, lstm1_bwd_w_hh, lstm1_bwd_b_ih, lstm1_bwd_b_hh,
           lstm2_fwd_w_ih, lstm2_fwd_w_hh, lstm2_fwd_b_ih, lstm2_fwd_b_hh,
           lstm2_bwd_w_ih, lstm2_bwd_w_hh, lstm2_bwd_b_ih, lstm2_bwd_b_hh,
           enc0_in_proj_w, enc0_in_proj_b, enc0_out_proj_w, enc0_out_proj_b,
           enc0_lin1_w, enc0_lin1_b, enc0_lin2_w, enc0_lin2_b,
           enc0_ln1_g, enc0_ln1_b, enc0_ln2_g, enc0_ln2_b,
           enc1_in_proj_w, enc1_in_proj_b, enc1_out_proj_w, enc1_out_proj_b,
           enc1_lin1_w, enc1_lin1_b, enc1_lin2_w, enc1_lin2_b,
           enc1_ln1_g, enc1_ln1_b, enc1_ln2_g, enc1_ln2_b,
           enc_norm_g, enc_norm_b, src):
    B, T, In = src.shape
    G = lstm0_fwd_w_hh.shape[1]
    H = 2 * G
    num_heads = 8

    def prep_lstm(w_ih_f, w_hh_f, b_ih_f, b_hh_f, w_ih_b, w_hh_b, b_ih_b, b_hh_b):
        wih = jnp.concatenate([w_ih_f.T, w_ih_b.T], axis=1)
        bias = jnp.concatenate([b_ih_f + b_hh_f,
                                b_ih_b + b_hh_b]).reshape(1, 8 * G)
        return (wih, bias, w_hh_f.T, w_hh_b.T)

    lparams = [
        prep_lstm(lstm0_fwd_w_ih, lstm0_fwd_w_hh, lstm0_fwd_b_ih, lstm0_fwd_b_hh,
                  lstm0_bwd_w_ih, lstm0_bwd_w_hh, lstm0_bwd_b_ih, lstm0_bwd_b_hh),
        prep_lstm(lstm1_fwd_w_ih, lstm1_fwd_w_hh, lstm1_fwd_b_ih, lstm1_fwd_b_hh,
                  lstm1_bwd_w_ih, lstm1_bwd_w_hh, lstm1_bwd_b_ih, lstm1_bwd_b_hh),
        prep_lstm(lstm2_fwd_w_ih, lstm2_fwd_w_hh, lstm2_fwd_b_ih, lstm2_fwd_b_hh,
                  lstm2_bwd_w_ih, lstm2_bwd_w_hh, lstm2_bwd_b_ih, lstm2_bwd_b_hh),
    ]

    def prep_enc(in_w, in_b, out_w, out_b, w1, b1, w2, b2,
                 ln1_g, ln1_b, ln2_g, ln2_b, lnf_g, lnf_b):
        lnp = jnp.stack([ln1_g, ln1_b, ln2_g, ln2_b, lnf_g, lnf_b], axis=0)
        return (in_w.T, in_b.reshape(1, 3 * H), out_w.T, out_b.reshape(1, H),
                w1.T, b1.reshape(1, H), w2.T, b2.reshape(1, H), lnp)

    eparams = [
        prep_enc(enc0_in_proj_w, enc0_in_proj_b, enc0_out_proj_w, enc0_out_proj_b,
                 enc0_lin1_w, enc0_lin1_b, enc0_lin2_w, enc0_lin2_b,
                 enc0_ln1_g, enc0_ln1_b, enc0_ln2_g, enc0_ln2_b,
                 enc_norm_g, enc_norm_b),
        prep_enc(enc1_in_proj_w, enc1_in_proj_b, enc1_out_proj_w, enc1_out_proj_b,
                 enc1_lin1_w, enc1_lin1_b, enc1_lin2_w, enc1_lin2_b,
                 enc1_ln1_g, enc1_ln1_b, enc1_ln2_g, enc1_ln2_b,
                 enc_norm_g, enc_norm_b),
    ]

    xT = src.transpose(1, 0, 2)
    h = _lstm3(xT, lparams, T, B, G)
    h = _enc2(h.reshape(T * B, H), eparams, num_heads, B)
    hidden = h.reshape(T, B, H).transpose(1, 0, 2)
    return jnp.float32(0.0), hidden


# speedup vs baseline: 1.6563x; 1.6563x over previous
import functools
import math

import jax
import jax.numpy as jnp
from jax.experimental import pallas as pl
from jax.experimental.pallas import tpu as pltpu


def _lstm3_kernel(x_ref,
                  wih0_ref, b0_ref, whhf0_ref, whhb0_ref,
                  wih1_ref, b1_ref, whhf1_ref, whhb1_ref,
                  wih2_ref, b2_ref, whhf2_ref, whhb2_ref,
                  o_ref, scr_ref, *, T, Bh, G):
    def cell(gates, c):
        i = jax.nn.sigmoid(gates[:, 0 * G:1 * G])
        f = jax.nn.sigmoid(gates[:, 1 * G:2 * G])
        g = jnp.tanh(gates[:, 2 * G:3 * G])
        o = jax.nn.sigmoid(gates[:, 3 * G:4 * G])
        c_new = f * c + i * g
        return o * jnp.tanh(c_new), c_new

    def layer(x2d, wih_ref, b_ref, whhf_ref, whhb_ref, out_ref, out_3d):
        xp = jnp.dot(x2d, wih_ref[...],
                     preferred_element_type=jnp.float32) + b_ref[...]
        whhf = whhf_ref[...]
        whhb = whhb_ref[...]
        h_f = jnp.zeros((Bh, G), jnp.float32)
        c_f = jnp.zeros((Bh, G), jnp.float32)
        h_b = jnp.zeros((Bh, G), jnp.float32)
        c_b = jnp.zeros((Bh, G), jnp.float32)
        for s in range(T):
            tf = s
            tb = T - 1 - s
            rec_f = jnp.dot(h_f, whhf, preferred_element_type=jnp.float32)
            rec_b = jnp.dot(h_b, whhb, preferred_element_type=jnp.float32)
            h_f, c_f = cell(xp[tf * Bh:(tf + 1) * Bh, 0:4 * G] + rec_f, c_f)
            h_b, c_b = cell(xp[tb * Bh:(tb + 1) * Bh, 4 * G:8 * G] + rec_b, c_b)
            if out_3d:
                out_ref[tf, :, 0:G] = h_f
                out_ref[tb, :, G:2 * G] = h_b
            else:
                out_ref[tf * Bh:(tf + 1) * Bh, 0:G] = h_f
                out_ref[tb * Bh:(tb + 1) * Bh, G:2 * G] = h_b

    In0 = x_ref.shape[-1]
    x2d = x_ref[...].reshape(T * Bh, In0)
    layer(x2d, wih0_ref, b0_ref, whhf0_ref, whhb0_ref, scr_ref, False)
    x2d = scr_ref[...]
    layer(x2d, wih1_ref, b1_ref, whhf1_ref, whhb1_ref, scr_ref, False)
    x2d = scr_ref[...]
    layer(x2d, wih2_ref, b2_ref, whhf2_ref, whhb2_ref, o_ref, True)


def _lstm3(xT, lparams, T, B, G):
    In0 = xT.shape[-1]
    flat = []
    wspecs = []
    for (wih, bias, whhf, whhb) in lparams:
        flat += [wih, bias, whhf, whhb]
        wspecs += [
            pl.BlockSpec(wih.shape, lambda i: (0, 0)),
            pl.BlockSpec(bias.shape, lambda i: (0, 0)),
            pl.BlockSpec(whhf.shape, lambda i: (0, 0)),
            pl.BlockSpec(whhb.shape, lambda i: (0, 0)),
        ]

    kern = functools.partial(_lstm3_kernel, T=T, Bh=B, G=G)
    return pl.pallas_call(
        kern,
        out_shape=jax.ShapeDtypeStruct((T, B, 2 * G), jnp.float32),
        grid=(1,),
        in_specs=[pl.BlockSpec((T, B, In0), lambda i: (0, 0, 0))] + wspecs,
        out_specs=pl.BlockSpec((T, B, 2 * G), lambda i: (0, 0, 0)),
        scratch_shapes=[pltpu.VMEM((T * B, 2 * G), jnp.float32)],
        compiler_params=pltpu.CompilerParams(
            dimension_semantics=("arbitrary",)),
    )(xT, *flat)


def _enc2_kernel(x_ref,
                 wqkv0_ref, bqkv0_ref, wo0_ref, bo0_ref,
                 w10_ref, b10_ref, w20_ref, b20_ref, ln0_ref,
                 wqkv1_ref, bqkv1_ref, wo1_ref, bo1_ref,
                 w11_ref, b11_ref, w21_ref, b21_ref, ln1_ref,
                 o_ref, *, num_heads, L, R):
    E = x_ref.shape[-1]
    d = E // num_heads
    scale = 1.0 / math.sqrt(d)

    if R == L:
        neg = None
    else:
        ri = jax.lax.broadcasted_iota(jnp.int32, (R, R), 0) // L
        ci = jax.lax.broadcasted_iota(jnp.int32, (R, R), 1) // L
        neg = jnp.where(ri == ci, 0.0, -1e9).astype(jnp.float32)

    def ln(z, g, b):
        mu = jnp.mean(z, axis=-1, keepdims=True)
        var = jnp.mean(jnp.square(z - mu), axis=-1, keepdims=True)
        return (z - mu) * jax.lax.rsqrt(var + 1e-5) * g + b

    def enc_layer(x, wqkv_ref, bqkv_ref, wo_ref, bo_ref,
                  w1_ref, b1_ref, w2_ref, b2_ref, ln_ref, final_norm):
        qkv = jnp.dot(x, wqkv_ref[...],
                      preferred_element_type=jnp.float32) + bqkv_ref[...]
        ctxs = []
        for hh in range(num_heads):
            qh = qkv[:, hh * d:(hh + 1) * d]
            kh = qkv[:, E + hh * d:E + (hh + 1) * d]
            vh = qkv[:, 2 * E + hh * d:2 * E + (hh + 1) * d]
            s = jnp.dot(qh, kh.T, preferred_element_type=jnp.float32) * scale
            if neg is not None:
                s = s + neg
            s = s - jnp.max(s, axis=-1, keepdims=True)
            p = jnp.exp(s)
            p = p * pl.reciprocal(jnp.sum(p, axis=-1, keepdims=True), approx=True)
            ctxs.append(jnp.dot(p, vh, preferred_element_type=jnp.float32))
        ctx = jnp.concatenate(ctxs, axis=1)
        attn = jnp.dot(ctx, wo_ref[...],
                       preferred_element_type=jnp.float32) + bo_ref[...]
        lnp = ln_ref[...]
        y = ln(x + attn, lnp[0:1, :], lnp[1:2, :])
        ff = jnp.maximum(
            jnp.dot(y, w1_ref[...],
                    preferred_element_type=jnp.float32) + b1_ref[...], 0.0)
        ff = jnp.dot(ff, w2_ref[...],
                     preferred_element_type=jnp.float32) + b2_ref[...]
        y = ln(y + ff, lnp[2:3, :], lnp[3:4, :])
        if final_norm:
            y = ln(y, lnp[4:5, :], lnp[5:6, :])
        return y

    x = x_ref[...]
    x = enc_layer(x, wqkv0_ref, bqkv0_ref, wo0_ref, bo0_ref,
                  w10_ref, b10_ref, w20_ref, b20_ref, ln0_ref, False)
    x = enc_layer(x, wqkv1_ref, bqkv1_ref, wo1_ref, bo1_ref,
                  w11_ref, b11_ref, w21_ref, b21_ref, ln1_ref, True)
    o_ref[...] = x


def _enc2(h2d, eparams, num_heads, L):
    M, E = h2d.shape
    R = min(M, 4 * L)
    n_blocks = M // R

    flat = []
    wspecs = []
    for p in eparams:
        for a in p:
            flat.append(a)
            wspecs.append(pl.BlockSpec(a.shape, lambda i: (0, 0)))

    kern = functools.partial(_enc2_kernel, num_heads=num_heads, L=L, R=R)
    return pl.pallas_call(
        kern,
        out_shape=jax.ShapeDtypeStruct((M, E), jnp.float32),
        grid=(n_blocks,),
        in_specs=[pl.BlockSpec((R, E), lambda i: (i, 0))] + wspecs,
        out_specs=pl.BlockSpec((R, E), lambda i: (i, 0)),
        compiler_params=pltpu.CompilerParams(
            dimension_semantics=("arbitrary",)),
    )(h2d, *flat)


def kernel(lstm0_fwd_w_ih, lstm0_fwd_w_hh, lstm0_fwd_b_ih, lstm0_fwd_b_hh,
           lstm0_bwd_w_ih, lstm0_bwd_w_hh, lstm0_bwd_b_ih, lstm0_bwd_b_hh,
           lstm1_fwd_w_ih, lstm1_fwd_w_hh, lstm1_fwd_b_ih, lstm1_fwd_b_hh,
           lstm1_bwd_w_ih, lstm1_bwd_w_hh, lstm1_bwd_b_ih, lstm1_bwd_b_hh,
           lstm2_fwd_w_ih, lstm2_fwd_w_hh, lstm2_fwd_b_ih, lstm2_fwd_b_hh,
           lstm2_bwd_w_ih, lstm2_bwd_w_hh, lstm2_bwd_b_ih, lstm2_bwd_b_hh,
           enc0_in_proj_w, enc0_in_proj_b, enc0_out_proj_w, enc0_out_proj_b,
           enc0_lin1_w, enc0_lin1_b, enc0_lin2_w, enc0_lin2_b,
           enc0_ln1_g, enc0_ln1_b, enc0_ln2_g, enc0_ln2_b,
           enc1_in_proj_w, enc1_in_proj_b, enc1_out_proj_w, enc1_out_proj_b,
           enc1_lin1_w, enc1_lin1_b, enc1_lin2_w, enc1_lin2_b,
           enc1_ln1_g, enc1_ln1_b, enc1_ln2_g, enc1_ln2_b,
           enc_norm_g, enc_norm_b, src):
    B, T, In = src.shape
    G = lstm0_fwd_w_hh.shape[1]
    H = 2 * G
    num_heads = 8

    def prep_lstm(w_ih_f, w_hh_f, b_ih_f, b_hh_f, w_ih_b, w_hh_b, b_ih_b, b_hh_b):
        wih = jnp.concatenate([w_ih_f.T, w_ih_b.T], axis=1)
        bias = jnp.concatenate([b_ih_f + b_hh_f,
                                b_ih_b + b_hh_b]).reshape(1, 8 * G)
        return (wih, bias, w_hh_f.T, w_hh_b.T)

    lparams = [
        prep_lstm(lstm0_fwd_w_ih, lstm0_fwd_w_hh, lstm0_fwd_b_ih, lstm0_fwd_b_hh,
                  lstm0_bwd_w_ih, lstm0_bwd_w_hh, lstm0_bwd_b_ih, lstm0_bwd_b_hh),
        prep_lstm(lstm1_fwd_w_ih, lstm1_fwd_w_hh, lstm1_fwd_b_ih, lstm1_fwd_b_hh,
                  lstm1_bwd_w_ih, lstm1_bwd_w_hh, lstm1_bwd_b_ih, lstm1_bwd_b_hh),
        prep_lstm(lstm2_fwd_w_ih, lstm2_fwd_w_hh, lstm2_fwd_b_ih, lstm2_fwd_b_hh,
                  lstm2_bwd_w_ih, lstm2_bwd_w_hh, lstm2_bwd_b_ih, lstm2_bwd_b_hh),
    ]

    def prep_enc(in_w, in_b, out_w, out_b, w1, b1, w2, b2,
                 ln1_g, ln1_b, ln2_g, ln2_b, lnf_g, lnf_b):
        lnp = jnp.stack([ln1_g, ln1_b, ln2_g, ln2_b, lnf_g, lnf_b], axis=0)
        return (in_w.T, in_b.reshape(1, 3 * H), out_w.T, out_b.reshape(1, H),
                w1.T, b1.reshape(1, H), w2.T, b2.reshape(1, H), lnp)

    eparams = [
        prep_enc(enc0_in_proj_w, enc0_in_proj_b, enc0_out_proj_w, enc0_out_proj_b,
                 enc0_lin1_w, enc0_lin1_b, enc0_lin2_w, enc0_lin2_b,
                 enc0_ln1_g, enc0_ln1_b, enc0_ln2_g, enc0_ln2_b,
                 enc_norm_g, enc_norm_b),
        prep_enc(enc1_in_proj_w, enc1_in_proj_b, enc1_out_proj_w, enc1_out_proj_b,
                 enc1_lin1_w, enc1_lin1_b, enc1_lin2_w, enc1_lin2_b,
                 enc1_ln1_g, enc1_ln1_b, enc1_ln2_g, enc1_ln2_b,
                 enc_norm_g, enc_norm_b),
    ]

    xT = src.transpose(1, 0, 2)
    h = _lstm3(xT, lparams, T, B, G)
    h = _enc2(h.reshape(T * B, H), eparams, num_heads, B)
    hidden = h.reshape(T, B, H).transpose(1, 0, 2)
    return jnp.float32(0.0), hidden


# speedup vs baseline: 2.0591x; 1.2432x over previous
import functools
import math

import jax
import jax.numpy as jnp
from jax.experimental import pallas as pl
from jax.experimental.pallas import tpu as pltpu


def _lstm3_kernel(x_ref,
                  wih0_ref, b0_ref, whhf0_ref, whhb0_ref,
                  wih1_ref, b1_ref, whhf1_ref, whhb1_ref,
                  wih2_ref, b2_ref, whhf2_ref, whhb2_ref,
                  o_ref, scr_ref, *, T, Bh, G):
    bf16 = jnp.bfloat16

    def cell(gates, c):
        i = jax.nn.sigmoid(gates[:, 0 * G:1 * G])
        f = jax.nn.sigmoid(gates[:, 1 * G:2 * G])
        g = jnp.tanh(gates[:, 2 * G:3 * G])
        o = jax.nn.sigmoid(gates[:, 3 * G:4 * G])
        c_new = f * c + i * g
        return o * jnp.tanh(c_new), c_new

    def layer(x2d, wih_ref, b_ref, whhf_ref, whhb_ref, out_ref, last):
        xp = jnp.dot(x2d, wih_ref[...],
                     preferred_element_type=jnp.float32) + b_ref[...]
        whhf = whhf_ref[...]
        whhb = whhb_ref[...]
        h_f = jnp.zeros((Bh, G), jnp.float32)
        c_f = jnp.zeros((Bh, G), jnp.float32)
        h_b = jnp.zeros((Bh, G), jnp.float32)
        c_b = jnp.zeros((Bh, G), jnp.float32)
        for s in range(T):
            tf = s
            tb = T - 1 - s
            rec_f = jnp.dot(h_f.astype(bf16), whhf,
                            preferred_element_type=jnp.float32)
            rec_b = jnp.dot(h_b.astype(bf16), whhb,
                            preferred_element_type=jnp.float32)
            h_f, c_f = cell(xp[tf * Bh:(tf + 1) * Bh, 0:4 * G] + rec_f, c_f)
            h_b, c_b = cell(xp[tb * Bh:(tb + 1) * Bh, 4 * G:8 * G] + rec_b, c_b)
            if last:
                out_ref[tf, :, 0:G] = h_f
                out_ref[tb, :, G:2 * G] = h_b
            else:
                out_ref[tf * Bh:(tf + 1) * Bh, 0:G] = h_f.astype(bf16)
                out_ref[tb * Bh:(tb + 1) * Bh, G:2 * G] = h_b.astype(bf16)

    In0 = x_ref.shape[-1]
    x2d = x_ref[...].reshape(T * Bh, In0).astype(jnp.bfloat16)
    layer(x2d, wih0_ref, b0_ref, whhf0_ref, whhb0_ref, scr_ref, False)
    layer(scr_ref[...], wih1_ref, b1_ref, whhf1_ref, whhb1_ref, scr_ref, False)
    layer(scr_ref[...], wih2_ref, b2_ref, whhf2_ref, whhb2_ref, o_ref, True)


def _lstm3(xT, lparams, T, B, G):
    In0 = xT.shape[-1]
    flat = []
    wspecs = []
    for (wih, bias, whhf, whhb) in lparams:
        flat += [wih, bias, whhf, whhb]
        wspecs += [
            pl.BlockSpec(wih.shape, lambda i: (0, 0)),
            pl.BlockSpec(bias.shape, lambda i: (0, 0)),
            pl.BlockSpec(whhf.shape, lambda i: (0, 0)),
            pl.BlockSpec(whhb.shape, lambda i: (0, 0)),
        ]

    kern = functools.partial(_lstm3_kernel, T=T, Bh=B, G=G)
    return pl.pallas_call(
        kern,
        out_shape=jax.ShapeDtypeStruct((T, B, 2 * G), jnp.float32),
        grid=(1,),
        in_specs=[pl.BlockSpec((T, B, In0), lambda i: (0, 0, 0))] + wspecs,
        out_specs=pl.BlockSpec((T, B, 2 * G), lambda i: (0, 0, 0)),
        scratch_shapes=[pltpu.VMEM((T * B, 2 * G), jnp.bfloat16)],
        compiler_params=pltpu.CompilerParams(
            dimension_semantics=("arbitrary",)),
    )(xT, *flat)


def _enc2_kernel(x_ref,
                 wqkv0_ref, bqkv0_ref, wo0_ref, bo0_ref,
                 w10_ref, b10_ref, w20_ref, b20_ref, ln0_ref,
                 wqkv1_ref, bqkv1_ref, wo1_ref, bo1_ref,
                 w11_ref, b11_ref, w21_ref, b21_ref, ln1_ref,
                 o_ref, *, num_heads, L, R):
    bf16 = jnp.bfloat16
    E = x_ref.shape[-1]
    d = E // num_heads
    scale = 1.0 / math.sqrt(d)

    if R == L:
        neg = None
    else:
        ri = jax.lax.broadcasted_iota(jnp.int32, (R, R), 0) // L
        ci = jax.lax.broadcasted_iota(jnp.int32, (R, R), 1) // L
        neg = jnp.where(ri == ci, 0.0, -1e9).astype(jnp.float32)

    def ln(z, g, b):
        mu = jnp.mean(z, axis=-1, keepdims=True)
        var = jnp.mean(jnp.square(z - mu), axis=-1, keepdims=True)
        return (z - mu) * jax.lax.rsqrt(var + 1e-5) * g + b

    def enc_layer(x, wqkv_ref, bqkv_ref, wo_ref, bo_ref,
                  w1_ref, b1_ref, w2_ref, b2_ref, ln_ref, final_norm):
        qkv = jnp.dot(x.astype(bf16), wqkv_ref[...],
                      preferred_element_type=jnp.float32) + bqkv_ref[...]
        qkv_b = qkv.astype(bf16)
        ctxs = []
        for hh in range(num_heads):
            qh = qkv_b[:, hh * d:(hh + 1) * d]
            kh = qkv_b[:, E + hh * d:E + (hh + 1) * d]
            vh = qkv_b[:, 2 * E + hh * d:2 * E + (hh + 1) * d]
            s = jnp.dot(qh, kh.T, preferred_element_type=jnp.float32) * scale
            if neg is not None:
                s = s + neg
            s = s - jnp.max(s, axis=-1, keepdims=True)
            p = jnp.exp(s)
            p = p * pl.reciprocal(jnp.sum(p, axis=-1, keepdims=True), approx=True)
            ctxs.append(jnp.dot(p.astype(bf16), vh,
                                preferred_element_type=jnp.float32))
        ctx = jnp.concatenate(ctxs, axis=1)
        attn = jnp.dot(ctx.astype(bf16), wo_ref[...],
                       preferred_element_type=jnp.float32) + bo_ref[...]
        lnp = ln_ref[...]
        y = ln(x + attn, lnp[0:1, :], lnp[1:2, :])
        ff = jnp.maximum(
            jnp.dot(y.astype(bf16), w1_ref[...],
                    preferred_element_type=jnp.float32) + b1_ref[...], 0.0)
        ff = jnp.dot(ff.astype(bf16), w2_ref[...],
                     preferred_element_type=jnp.float32) + b2_ref[...]
        y = ln(y + ff, lnp[2:3, :], lnp[3:4, :])
        if final_norm:
            y = ln(y, lnp[4:5, :], lnp[5:6, :])
        return y

    x = x_ref[...]
    x = enc_layer(x, wqkv0_ref, bqkv0_ref, wo0_ref, bo0_ref,
                  w10_ref, b10_ref, w20_ref, b20_ref, ln0_ref, False)
    x = enc_layer(x, wqkv1_ref, bqkv1_ref, wo1_ref, bo1_ref,
                  w11_ref, b11_ref, w21_ref, b21_ref, ln1_ref, True)
    o_ref[...] = x


def _enc2(h2d, eparams, num_heads, L):
    M, E = h2d.shape
    R = min(M, 4 * L)
    n_blocks = M // R

    flat = []
    wspecs = []
    for p in eparams:
        for a in p:
            flat.append(a)
            wspecs.append(pl.BlockSpec(a.shape, lambda i: (0, 0)))

    kern = functools.partial(_enc2_kernel, num_heads=num_heads, L=L, R=R)
    return pl.pallas_call(
        kern,
        out_shape=jax.ShapeDtypeStruct((M, E), jnp.float32),
        grid=(n_blocks,),
        in_specs=[pl.BlockSpec((R, E), lambda i: (i, 0))] + wspecs,
        out_specs=pl.BlockSpec((R, E), lambda i: (i, 0)),
        compiler_params=pltpu.CompilerParams(
            dimension_semantics=("arbitrary",)),
    )(h2d, *flat)


def kernel(lstm0_fwd_w_ih, lstm0_fwd_w_hh, lstm0_fwd_b_ih, lstm0_fwd_b_hh,
           lstm0_bwd_w_ih, lstm0_bwd_w_hh, lstm0_bwd_b_ih, lstm0_bwd_b_hh,
           lstm1_fwd_w_ih, lstm1_fwd_w_hh, lstm1_fwd_b_ih, lstm1_fwd_b_hh,
           lstm1_bwd_w_ih, lstm1_bwd_w_hh, lstm1_bwd_b_ih, lstm1_bwd_b_hh,
           lstm2_fwd_w_ih, lstm2_fwd_w_hh, lstm2_fwd_b_ih, lstm2_fwd_b_hh,
           lstm2_bwd_w_ih, lstm2_bwd_w_hh, lstm2_bwd_b_ih, lstm2_bwd_b_hh,
           enc0_in_proj_w, enc0_in_proj_b, enc0_out_proj_w, enc0_out_proj_b,
           enc0_lin1_w, enc0_lin1_b, enc0_lin2_w, enc0_lin2_b,
           enc0_ln1_g, enc0_ln1_b, enc0_ln2_g, enc0_ln2_b,
           enc1_in_proj_w, enc1_in_proj_b, enc1_out_proj_w, enc1_out_proj_b,
           enc1_lin1_w, enc1_lin1_b, enc1_lin2_w, enc1_lin2_b,
           enc1_ln1_g, enc1_ln1_b, enc1_ln2_g, enc1_ln2_b,
           enc_norm_g, enc_norm_b, src):
    B, T, In = src.shape
    G = lstm0_fwd_w_hh.shape[1]
    H = 2 * G
    num_heads = 8
    bf16 = jnp.bfloat16

    def prep_lstm(w_ih_f, w_hh_f, b_ih_f, b_hh_f, w_ih_b, w_hh_b, b_ih_b, b_hh_b):
        wih = jnp.concatenate([w_ih_f.T, w_ih_b.T], axis=1).astype(bf16)
        bias = jnp.concatenate([b_ih_f + b_hh_f,
                                b_ih_b + b_hh_b]).reshape(1, 8 * G)
        return (wih, bias, w_hh_f.T.astype(bf16), w_hh_b.T.astype(bf16))

    lparams = [
        prep_lstm(lstm0_fwd_w_ih, lstm0_fwd_w_hh, lstm0_fwd_b_ih, lstm0_fwd_b_hh,
                  lstm0_bwd_w_ih, lstm0_bwd_w_hh, lstm0_bwd_b_ih, lstm0_bwd_b_hh),
        prep_lstm(lstm1_fwd_w_ih, lstm1_fwd_w_hh, lstm1_fwd_b_ih, lstm1_fwd_b_hh,
                  lstm1_bwd_w_ih, lstm1_bwd_w_hh, lstm1_bwd_b_ih, lstm1_bwd_b_hh),
        prep_lstm(lstm2_fwd_w_ih, lstm2_fwd_w_hh, lstm2_fwd_b_ih, lstm2_fwd_b_hh,
                  lstm2_bwd_w_ih, lstm2_bwd_w_hh, lstm2_bwd_b_ih, lstm2_bwd_b_hh),
    ]

    def prep_enc(in_w, in_b, out_w, out_b, w1, b1, w2, b2,
                 ln1_g, ln1_b, ln2_g, ln2_b, lnf_g, lnf_b):
        lnp = jnp.stack([ln1_g, ln1_b, ln2_g, ln2_b, lnf_g, lnf_b], axis=0)
        return (in_w.T.astype(bf16), in_b.reshape(1, 3 * H),
                out_w.T.astype(bf16), out_b.reshape(1, H),
                w1.T.astype(bf16), b1.reshape(1, H),
                w2.T.astype(bf16), b2.reshape(1, H), lnp)

    eparams = [
        prep_enc(enc0_in_proj_w, enc0_in_proj_b, enc0_out_proj_w, enc0_out_proj_b,
                 enc0_lin1_w, enc0_lin1_b, enc0_lin2_w, enc0_lin2_b,
                 enc0_ln1_g, enc0_ln1_b, enc0_ln2_g, enc0_ln2_b,
                 enc_norm_g, enc_norm_b),
        prep_enc(enc1_in_proj_w, enc1_in_proj_b, enc1_out_proj_w, enc1_out_proj_b,
                 enc1_lin1_w, enc1_lin1_b, enc1_lin2_w, enc1_lin2_b,
                 enc1_ln1_g, enc1_ln1_b, enc1_ln2_g, enc1_ln2_b,
                 enc_norm_g, enc_norm_b),
    ]

    xT = src.transpose(1, 0, 2)
    h = _lstm3(xT, lparams, T, B, G)
    h = _enc2(h.reshape(T * B, H), eparams, num_heads, B)
    hidden = h.reshape(T, B, H).transpose(1, 0, 2)
    return jnp.float32(0.0), hidden


# speedup vs baseline: 2.4144x; 1.1725x over previous
import functools
import math

import jax
import jax.numpy as jnp
from jax.experimental import pallas as pl
from jax.experimental.pallas import tpu as pltpu

_BF16 = jnp.bfloat16


def _dot_t(a, w):
    return jax.lax.dot_general(a, w, (((1,), (1,)), ((), ())),
                               preferred_element_type=jnp.float32)


def _lstm3_kernel(x_ref,
                  wihf0_ref, wihb0_ref, whhf0_ref, whhb0_ref, b0_ref,
                  wihf1_ref, wihb1_ref, whhf1_ref, whhb1_ref, b1_ref,
                  wihf2_ref, wihb2_ref, whhf2_ref, whhb2_ref, b2_ref,
                  o_ref, scr_ref, *, T, B, G):
    def cell(gates, c):
        i = jax.nn.sigmoid(gates[:, 0 * G:1 * G])
        f = jax.nn.sigmoid(gates[:, 1 * G:2 * G])
        g = jnp.tanh(gates[:, 2 * G:3 * G])
        o = jax.nn.sigmoid(gates[:, 3 * G:4 * G])
        c_new = f * c + i * g
        return o * jnp.tanh(c_new), c_new

    def layer(x2d, wihf_ref, wihb_ref, whhf_ref, whhb_ref, b_ref, out_ref, last):
        xpf = _dot_t(x2d, wihf_ref[...].astype(_BF16)) + b_ref[0:1, 0:4 * G]
        xpb = _dot_t(x2d, wihb_ref[...].astype(_BF16)) + b_ref[0:1, 4 * G:8 * G]
        whhf = whhf_ref[...].astype(_BF16)
        whhb = whhb_ref[...].astype(_BF16)
        h_f = jnp.zeros((B, G), jnp.float32)
        c_f = jnp.zeros((B, G), jnp.float32)
        h_b = jnp.zeros((B, G), jnp.float32)
        c_b = jnp.zeros((B, G), jnp.float32)
        for s in range(T):
            tf = s
            tb = T - 1 - s
            rec_f = _dot_t(h_f.astype(_BF16), whhf)
            rec_b = _dot_t(h_b.astype(_BF16), whhb)
            h_f, c_f = cell(xpf[tf * B:(tf + 1) * B, :] + rec_f, c_f)
            h_b, c_b = cell(xpb[tb * B:(tb + 1) * B, :] + rec_b, c_b)
            if last:
                out_ref[tf, :, 0:G] = h_f
                out_ref[tb, :, G:2 * G] = h_b
            else:
                out_ref[tf * B:(tf + 1) * B, 0:G] = h_f.astype(_BF16)
                out_ref[tb * B:(tb + 1) * B, G:2 * G] = h_b.astype(_BF16)

    In0 = x_ref.shape[-1]
    x2d = jnp.transpose(x_ref[...], (1, 0, 2)).reshape(T * B, In0).astype(_BF16)
    layer(x2d, wihf0_ref, wihb0_ref, whhf0_ref, whhb0_ref, b0_ref, scr_ref, False)
    layer(scr_ref[...], wihf1_ref, wihb1_ref, whhf1_ref, whhb1_ref, b1_ref,
          scr_ref, False)
    layer(scr_ref[...], wihf2_ref, wihb2_ref, whhf2_ref, whhb2_ref, b2_ref,
          o_ref, True)


def _lstm3(src, lweights, lbiases, T, B, G):
    In0 = src.shape[-1]
    flat = []
    wspecs = []
    for (wihf, wihb, whhf, whhb), bias in zip(lweights, lbiases):
        flat += [wihf, wihb, whhf, whhb, bias]
        for a in (wihf, wihb, whhf, whhb, bias):
            wspecs.append(pl.BlockSpec(a.shape, lambda i, n=a.ndim: (0,) * n))

    kern = functools.partial(_lstm3_kernel, T=T, B=B, G=G)
    return pl.pallas_call(
        kern,
        out_shape=jax.ShapeDtypeStruct((T, B, 2 * G), jnp.float32),
        grid=(1,),
        in_specs=[pl.BlockSpec((B, T, In0), lambda i: (0, 0, 0))] + wspecs,
        out_specs=pl.BlockSpec((T, B, 2 * G), lambda i: (0, 0, 0)),
        scratch_shapes=[pltpu.VMEM((T * B, 2 * G), _BF16)],
        compiler_params=pltpu.CompilerParams(
            dimension_semantics=("arbitrary",)),
    )(src, *flat)


def _enc2_kernel(x_ref,
                 wqkv0_ref, wo0_ref, w10_ref, w20_ref, bias0_ref, ln0_ref,
                 wqkv1_ref, wo1_ref, w11_ref, w21_ref, bias1_ref, ln1_ref,
                 o_ref, *, num_heads, L, R):
    E = x_ref.shape[-1]
    d = E // num_heads
    scale = 1.0 / math.sqrt(d)

    if R == L:
        neg = None
    else:
        ri = jax.lax.broadcasted_iota(jnp.int32, (R, R), 0) // L
        ci = jax.lax.broadcasted_iota(jnp.int32, (R, R), 1) // L
        neg = jnp.where(ri == ci, 0.0, -1e9).astype(jnp.float32)

    def ln(z, g, b):
        mu = jnp.mean(z, axis=-1, keepdims=True)
        var = jnp.mean(jnp.square(z - mu), axis=-1, keepdims=True)
        return (z - mu) * jax.lax.rsqrt(var + 1e-5) * g + b

    def enc_layer(x, wqkv_ref, wo_ref, w1_ref, w2_ref, bias_ref, ln_ref,
                  final_norm):
        bias = bias_ref[...]
        qkv = _dot_t(x.astype(_BF16), wqkv_ref[...].astype(_BF16)) \
            + bias[0:1, 0:3 * E]
        qkv_b = qkv.astype(_BF16)
        ctxs = []
        for hh in range(num_heads):
            qh = qkv_b[:, hh * d:(hh + 1) * d]
            kh = qkv_b[:, E + hh * d:E + (hh + 1) * d]
            vh = qkv_b[:, 2 * E + hh * d:2 * E + (hh + 1) * d]
            s = _dot_t(qh, kh) * scale
            if neg is not None:
                s = s + neg
            s = s - jnp.max(s, axis=-1, keepdims=True)
            p = jnp.exp(s)
            p = p * pl.reciprocal(jnp.sum(p, axis=-1, keepdims=True), approx=True)
            ctxs.append(jnp.dot(p.astype(_BF16), vh,
                                preferred_element_type=jnp.float32))
        ctx = jnp.concatenate(ctxs, axis=1)
        attn = _dot_t(ctx.astype(_BF16), wo_ref[...].astype(_BF16)) \
            + bias[0:1, 3 * E:4 * E]
        lnp = ln_ref[...]
        y = ln(x + attn, lnp[0:1, :], lnp[1:2, :])
        ff = jnp.maximum(
            _dot_t(y.astype(_BF16), w1_ref[...].astype(_BF16))
            + bias[0:1, 4 * E:5 * E], 0.0)
        ff = _dot_t(ff.astype(_BF16), w2_ref[...].astype(_BF16)) \
            + bias[0:1, 5 * E:6 * E]
        y = ln(y + ff, lnp[2:3, :], lnp[3:4, :])
        if final_norm:
            y = ln(y, lnp[4:5, :], lnp[5:6, :])
        return y

    x = x_ref[...]
    x = enc_layer(x, wqkv0_ref, wo0_ref, w10_ref, w20_ref, bias0_ref,
                  ln0_ref, False)
    x = enc_layer(x, wqkv1_ref, wo1_ref, w11_ref, w21_ref, bias1_ref,
                  ln1_ref, True)
    o_ref[...] = x


def _enc2(h2d, eparams, num_heads, L):
    M, E = h2d.shape
    R = min(M, 4 * L)
    n_blocks = M // R

    flat = []
    wspecs = []
    for p in eparams:
        for a in p:
            flat.append(a)
            wspecs.append(pl.BlockSpec(a.shape, lambda i: (0, 0)))

    kern = functools.partial(_enc2_kernel, num_heads=num_heads, L=L, R=R)
    return pl.pallas_call(
        kern,
        out_shape=jax.ShapeDtypeStruct((M, E), jnp.float32),
        grid=(n_blocks,),
        in_specs=[pl.BlockSpec((R, E), lambda i: (i, 0))] + wspecs,
        out_specs=pl.BlockSpec((R, E), lambda i: (i, 0)),
        compiler_params=pltpu.CompilerParams(
            dimension_semantics=("arbitrary",)),
    )(h2d, *flat)


def kernel(lstm0_fwd_w_ih, lstm0_fwd_w_hh, lstm0_fwd_b_ih, lstm0_fwd_b_hh,
           lstm0_bwd_w_ih, lstm0_bwd_w_hh, lstm0_bwd_b_ih, lstm0_bwd_b_hh,
           lstm1_fwd_w_ih, lstm1_fwd_w_hh, lstm1_fwd_b_ih, lstm1_fwd_b_hh,
           lstm1_bwd_w_ih, lstm1_bwd_w_hh, lstm1_bwd_b_ih, lstm1_bwd_b_hh,
           lstm2_fwd_w_ih, lstm2_fwd_w_hh, lstm2_fwd_b_ih, lstm2_fwd_b_hh,
           lstm2_bwd_w_ih, lstm2_bwd_w_hh, lstm2_bwd_b_ih, lstm2_bwd_b_hh,
           enc0_in_proj_w, enc0_in_proj_b, enc0_out_proj_w, enc0_out_proj_b,
           enc0_lin1_w, enc0_lin1_b, enc0_lin2_w, enc0_lin2_b,
           enc0_ln1_g, enc0_ln1_b, enc0_ln2_g, enc0_ln2_b,
           enc1_in_proj_w, enc1_in_proj_b, enc1_out_proj_w, enc1_out_proj_b,
           enc1_lin1_w, enc1_lin1_b, enc1_lin2_w, enc1_lin2_b,
           enc1_ln1_g, enc1_ln1_b, enc1_ln2_g, enc1_ln2_b,
           enc_norm_g, enc_norm_b, src):
    B, T, In = src.shape
    G = lstm0_fwd_w_hh.shape[1]
    H = 2 * G
    num_heads = 8

    lweights = [
        (lstm0_fwd_w_ih, lstm0_bwd_w_ih, lstm0_fwd_w_hh, lstm0_bwd_w_hh),
        (lstm1_fwd_w_ih, lstm1_bwd_w_ih, lstm1_fwd_w_hh, lstm1_bwd_w_hh),
        (lstm2_fwd_w_ih, lstm2_bwd_w_ih, lstm2_fwd_w_hh, lstm2_bwd_w_hh),
    ]
    lbiases = [
        jnp.concatenate([lstm0_fwd_b_ih + lstm0_fwd_b_hh,
                         lstm0_bwd_b_ih + lstm0_bwd_b_hh]).reshape(1, 8 * G),
        jnp.concatenate([lstm1_fwd_b_ih + lstm1_fwd_b_hh,
                         lstm1_bwd_b_ih + lstm1_bwd_b_hh]).reshape(1, 8 * G),
        jnp.concatenate([lstm2_fwd_b_ih + lstm2_fwd_b_hh,
                         lstm2_bwd_b_ih + lstm2_bwd_b_hh]).reshape(1, 8 * G),
    ]

    eparams = [
        (enc0_in_proj_w, enc0_out_proj_w, enc0_lin1_w, enc0_lin2_w,
         jnp.concatenate([enc0_in_proj_b, enc0_out_proj_b,
                          enc0_lin1_b, enc0_lin2_b]).reshape(1, 6 * H),
         jnp.stack([enc0_ln1_g, enc0_ln1_b, enc0_ln2_g, enc0_ln2_b,
                    enc_norm_g, enc_norm_b], axis=0)),
        (enc1_in_proj_w, enc1_out_proj_w, enc1_lin1_w, enc1_lin2_w,
         jnp.concatenate([enc1_in_proj_b, enc1_out_proj_b,
                          enc1_lin1_b, enc1_lin2_b]).reshape(1, 6 * H),
         jnp.stack([enc1_ln1_g, enc1_ln1_b, enc1_ln2_g, enc1_ln2_b,
                    enc_norm_g, enc_norm_b], axis=0)),
    ]

    h = _lstm3(src, lweights, lbiases, T, B, G)
    h = _enc2(h.reshape(T * B, H), eparams, num_heads, B)
    hidden = h.reshape(T, B, H).transpose(1, 0, 2)
    return jnp.float32(0.0), hidden
